# baseline (speedup 1.0000x reference)
import jax
import jax.numpy as jnp
from jax import lax
from jax.experimental import pallas as pl
from jax.experimental.pallas import tpu as pltpu
from jax.experimental.pallas import tpu_sc as plsc

N = 10000
E = 320000
D = 128
LANES = 16

NUM_CORES = 2
NUM_TILES = 16
NUM_WORKERS = NUM_CORES * NUM_TILES
EPW = E // NUM_WORKERS
K = 128
EPW_PAD = 10240
CHUNKS = EPW_PAD // K
ROWS_PER_TILE = 624
OUT_CHUNK = 104
OUT_CHUNKS = ROWS_PER_TILE // OUT_CHUNK
TAIL_BASE = NUM_TILES * ROWS_PER_TILE
TAIL_ROWS = N - TAIL_BASE


def _body(pidx_hbm, pval_hbm, feat_hbm, out_hbm,
          acc, ebuf0, ebuf1, vbuf0, vbuf1, ridx0, ridx1,
          rows0, rows1, obuf, sem_e0, sem_e1, sem_g0, sem_g1, sem_s0, sem_s1):
    cid = lax.axis_index("c")
    sid = lax.axis_index("s")
    wid = cid * NUM_TILES + sid
    chunk_base = wid * CHUNKS

    def zrow(r, c):
        for j in range(D // LANES):
            obuf[r, pl.ds(LANES * j, LANES)] = jnp.zeros((LANES,), jnp.float32)
        return c
    lax.fori_loop(0, OUT_CHUNK, zrow, 0)
    row_base = sid * ROWS_PER_TILE
    for c in range(OUT_CHUNKS):
        pltpu.sync_copy(obuf, acc.at[pl.ds(row_base + c * OUT_CHUNK, OUT_CHUNK)])

    @pl.when(sid == NUM_TILES - 1)
    def _():
        pltpu.sync_copy(obuf.at[pl.ds(0, TAIL_ROWS)],
                        acc.at[pl.ds(TAIL_BASE, TAIL_ROWS)])
    plsc.subcore_barrier()

    def e_start(c, ebuf, vbuf, sem_e):
        pltpu.async_copy(pidx_hbm.at[chunk_base + c], ebuf, sem_e)
        pltpu.async_copy(pval_hbm.at[chunk_base + c], vbuf, sem_e)

    def e_wait(c, ebuf, vbuf, sem_e):
        pltpu.make_async_copy(pidx_hbm.at[chunk_base + c], ebuf, sem_e).wait()
        pltpu.make_async_copy(pval_hbm.at[chunk_base + c], vbuf, sem_e).wait()

    def g_start(ebuf, rows_v, sem_g):
        pltpu.async_copy(feat_hbm.at[ebuf.at[0]], rows_v, sem_g)

    def g_wait(ebuf, rows_v, sem_g):
        pltpu.make_async_copy(feat_hbm.at[ebuf.at[0]], rows_v, sem_g).wait()

    def s_start(ridx_v, rows_v, sem_s):
        pltpu.async_copy(rows_v, acc.at[ridx_v], sem_s, add=True)

    def s_wait(ridx_v, rows_v, sem_s):
        pltpu.make_async_copy(rows_v, acc.at[ridx_v], sem_s).wait()

    def scale(vbuf, rows_v):
        def e_body(t, cc):
            vv = vbuf[pl.ds(t * LANES, LANES)]
            for i in range(LANES):
                e = t * LANES + i
                v = vv[i]
                for j in range(D // LANES):
                    rows_v[e, pl.ds(LANES * j, LANES)] = (
                        rows_v[e, pl.ds(LANES * j, LANES)] * v)
            return cc
        lax.fori_loop(0, K // LANES, e_body, 0)

    bufs0 = (ebuf0, vbuf0, ridx0, rows0, sem_e0, sem_g0, sem_s0)
    bufs1 = (ebuf1, vbuf1, ridx1, rows1, sem_e1, sem_g1, sem_s1)

    def phase(c, A, B, drain_prev, prefetch_next):
        ebuf_a, vbuf_a, ridx_a, rows_a, sem_ea, sem_ga, sem_sa = A
        ebuf_b, vbuf_b, ridx_b, rows_b, sem_eb, sem_gb, sem_sb = B
        g_wait(ebuf_a, rows_a, sem_ga)
        if drain_prev:
            s_wait(ridx_b, rows_b, sem_sb)
        if prefetch_next:
            e_start(c + 1, ebuf_b, vbuf_b, sem_eb)
        for t in range(K // LANES):
            ridx_a[pl.ds(t * LANES, LANES)] = ebuf_a[1, pl.ds(t * LANES, LANES)]
        scale(vbuf_a, rows_a)
        if prefetch_next:
            e_wait(c + 1, ebuf_b, vbuf_b, sem_eb)
            g_start(ebuf_b, rows_b, sem_gb)
        s_start(ridx_a, rows_a, sem_sa)

    e_start(0, ebuf0, vbuf0, sem_e0)
    e_wait(0, ebuf0, vbuf0, sem_e0)
    g_start(ebuf0, rows0, sem_g0)
    phase(0, bufs0, bufs1, False, True)
    phase(1, bufs1, bufs0, True, True)

    def pair_body(p, carry):
        c0 = 2 * p + 2
        phase(c0, bufs0, bufs1, True, True)
        phase(c0 + 1, bufs1, bufs0, True, True)
        return carry

    lax.fori_loop(0, (CHUNKS - 4) // 2, pair_body, 0)
    phase(CHUNKS - 2, bufs0, bufs1, True, True)
    phase(CHUNKS - 1, bufs1, bufs0, True, False)
    s_wait(ridx1, rows1, sem_s1)
    plsc.subcore_barrier()

    for c in range(OUT_CHUNKS):
        sl = pl.ds(row_base + c * OUT_CHUNK, OUT_CHUNK)
        pltpu.sync_copy(acc.at[sl], obuf)
        pltpu.sync_copy(obuf, out_hbm.at[cid].at[sl])

    @pl.when(sid == NUM_TILES - 1)
    def _():
        sl = pl.ds(TAIL_BASE, TAIL_ROWS)
        pltpu.sync_copy(acc.at[sl], obuf.at[pl.ds(0, TAIL_ROWS)])
        pltpu.sync_copy(obuf.at[pl.ds(0, TAIL_ROWS)], out_hbm.at[cid].at[sl])


def _add_body(a_ref, b_ref, o_ref):
    o_ref[...] = a_ref[...] + b_ref[...]


def kernel(adj_indices, adj_values, feature):
    row = adj_indices[0]
    col = adj_indices[1]
    pad = EPW_PAD - EPW
    colp = jnp.pad(col.reshape(NUM_WORKERS, EPW), ((0, 0), (0, pad)))
    rowp = jnp.pad(row.reshape(NUM_WORKERS, EPW), ((0, 0), (0, pad)))
    valp = jnp.pad(adj_values.reshape(NUM_WORKERS, EPW), ((0, 0), (0, pad)))
    pidx = jnp.stack(
        [colp.reshape(NUM_WORKERS, CHUNKS, K),
         rowp.reshape(NUM_WORKERS, CHUNKS, K)],
        axis=2).reshape(NUM_WORKERS * CHUNKS, 2, K)
    pval = valp.reshape(NUM_WORKERS * CHUNKS, K)

    mesh = plsc.VectorSubcoreMesh(
        core_axis_name="c", subcore_axis_name="s", num_cores=NUM_CORES)
    k = pl.kernel(
        _body,
        out_type=jax.ShapeDtypeStruct((NUM_CORES, N, D), jnp.float32),
        mesh=mesh,
        scratch_types=[
            pltpu.VMEM_SHARED((N, D), jnp.float32),
            pltpu.VMEM((2, K), jnp.int32),
            pltpu.VMEM((2, K), jnp.int32),
            pltpu.VMEM((K,), jnp.float32),
            pltpu.VMEM((K,), jnp.float32),
            pltpu.VMEM((K,), jnp.int32),
            pltpu.VMEM((K,), jnp.int32),
            pltpu.VMEM((K, D), jnp.float32),
            pltpu.VMEM((K, D), jnp.float32),
            pltpu.VMEM((OUT_CHUNK, D), jnp.float32),
            pltpu.SemaphoreType.DMA,
            pltpu.SemaphoreType.DMA,
            pltpu.SemaphoreType.DMA,
            pltpu.SemaphoreType.DMA,
            pltpu.SemaphoreType.DMA,
            pltpu.SemaphoreType.DMA,
        ],
    )
    out2 = k(pidx, pval, feature)

    blk = 2000
    return pl.pallas_call(
        _add_body,
        out_shape=jax.ShapeDtypeStruct((N, D), jnp.float32),
        grid=(N // blk,),
        in_specs=[pl.BlockSpec((blk, D), lambda i: (i, 0)),
                  pl.BlockSpec((blk, D), lambda i: (i, 0))],
        out_specs=pl.BlockSpec((blk, D), lambda i: (i, 0)),
    )(out2[0], out2[1])


# speedup vs baseline: 1.1045x; 1.1045x over previous
import jax
import jax.numpy as jnp
from jax import lax
from jax.experimental import pallas as pl
from jax.experimental.pallas import tpu as pltpu
from jax.experimental.pallas import tpu_sc as plsc

N = 10000
E = 320000
D = 128
LANES = 16

NUM_CORES = 2
NUM_TILES = 16
NUM_WORKERS = NUM_CORES * NUM_TILES
EPW = E // NUM_WORKERS
K = 128
EPW_PAD = 10240
CHUNKS = EPW_PAD // K
ROWS_PER_TILE = 624
OUT_CHUNK = 104
OUT_CHUNKS = ROWS_PER_TILE // OUT_CHUNK
TAIL_BASE = NUM_TILES * ROWS_PER_TILE
TAIL_ROWS = N - TAIL_BASE


def _body(pidx_hbm, pval_hbm, feat_hbm, out_hbm,
          acc, ebuf0, ebuf1, vbuf0, vbuf1, idxv0, idxv1, ridx0, ridx1,
          rows0, rows1, obuf, sem_e0, sem_e1, sem_g0, sem_g1, sem_s0, sem_s1):
    cid = lax.axis_index("c")
    sid = lax.axis_index("s")
    wid = cid * NUM_TILES + sid
    chunk_base = wid * CHUNKS

    def zrow(r, c):
        for j in range(D // LANES):
            obuf[r, pl.ds(LANES * j, LANES)] = jnp.zeros((LANES,), jnp.float32)
        return c
    lax.fori_loop(0, OUT_CHUNK, zrow, 0)
    row_base = sid * ROWS_PER_TILE
    for c in range(OUT_CHUNKS):
        pltpu.sync_copy(obuf, acc.at[pl.ds(row_base + c * OUT_CHUNK, OUT_CHUNK)])

    @pl.when(sid == NUM_TILES - 1)
    def _():
        pltpu.sync_copy(obuf.at[pl.ds(0, TAIL_ROWS)],
                        acc.at[pl.ds(TAIL_BASE, TAIL_ROWS)])
    plsc.subcore_barrier()

    def e_start(c, ebuf, vbuf, sem_e):
        pltpu.async_copy(pidx_hbm.at[chunk_base + c], ebuf, sem_e)
        pltpu.async_copy(pval_hbm.at[chunk_base + c], vbuf, sem_e)

    def e_wait(c, ebuf, vbuf, sem_e):
        pltpu.make_async_copy(pidx_hbm.at[chunk_base + c], ebuf, sem_e).wait()
        pltpu.make_async_copy(pval_hbm.at[chunk_base + c], vbuf, sem_e).wait()

    def g_start(idx_v, rows_v, sem_g):
        pltpu.async_copy(feat_hbm.at[idx_v], rows_v, sem_g)

    def g_wait(idx_v, rows_v, sem_g):
        pltpu.make_async_copy(feat_hbm.at[idx_v], rows_v, sem_g).wait()

    def s_start(ridx_v, rows_v, sem_s):
        pltpu.async_copy(rows_v, acc.at[ridx_v], sem_s, add=True)

    def s_wait(ridx_v, rows_v, sem_s):
        pltpu.make_async_copy(rows_v, acc.at[ridx_v], sem_s).wait()

    def scale(vbuf, rows_v):
        def e_body(t, cc):
            vv = vbuf[pl.ds(t * LANES, LANES)]
            for i in range(LANES):
                e = t * LANES + i
                v = vv[i]
                for j in range(D // LANES):
                    rows_v[e, pl.ds(LANES * j, LANES)] = (
                        rows_v[e, pl.ds(LANES * j, LANES)] * v)
            return cc
        lax.fori_loop(0, K // LANES, e_body, 0)

    bufs0 = (ebuf0, vbuf0, idxv0, ridx0, rows0, sem_e0, sem_g0, sem_s0)
    bufs1 = (ebuf1, vbuf1, idxv1, ridx1, rows1, sem_e1, sem_g1, sem_s1)

    def phase(c, A, B, drain_prev, prefetch_next):
        ebuf_a, vbuf_a, idxv_a, ridx_a, rows_a, sem_ea, sem_ga, sem_sa = A
        ebuf_b, vbuf_b, idxv_b, ridx_b, rows_b, sem_eb, sem_gb, sem_sb = B
        g_wait(idxv_a, rows_a, sem_ga)
        if drain_prev:
            s_wait(ridx_b, rows_b, sem_sb)
        if prefetch_next:
            e_start(c + 1, ebuf_b, vbuf_b, sem_eb)
        for t in range(K // LANES):
            ridx_a[pl.ds(t * LANES, LANES)] = ebuf_a[1, pl.ds(t * LANES, LANES)]
        scale(vbuf_a, rows_a)
        if prefetch_next:
            e_wait(c + 1, ebuf_b, vbuf_b, sem_eb)
            for t in range(K // LANES):
                idxv_b[pl.ds(t * LANES, LANES)] = ebuf_b[0, pl.ds(t * LANES, LANES)]
            g_start(idxv_b, rows_b, sem_gb)
        s_start(ridx_a, rows_a, sem_sa)

    e_start(0, ebuf0, vbuf0, sem_e0)
    e_wait(0, ebuf0, vbuf0, sem_e0)
    for t in range(K // LANES):
        idxv0[pl.ds(t * LANES, LANES)] = ebuf0[0, pl.ds(t * LANES, LANES)]
    g_start(idxv0, rows0, sem_g0)
    phase(0, bufs0, bufs1, False, True)
    phase(1, bufs1, bufs0, True, True)

    def pair_body(p, carry):
        c0 = 2 * p + 2
        phase(c0, bufs0, bufs1, True, True)
        phase(c0 + 1, bufs1, bufs0, True, True)
        return carry

    lax.fori_loop(0, (CHUNKS - 4) // 2, pair_body, 0)
    phase(CHUNKS - 2, bufs0, bufs1, True, True)
    phase(CHUNKS - 1, bufs1, bufs0, True, False)
    s_wait(ridx1, rows1, sem_s1)
    plsc.subcore_barrier()

    for c in range(OUT_CHUNKS):
        sl = pl.ds(row_base + c * OUT_CHUNK, OUT_CHUNK)
        pltpu.sync_copy(acc.at[sl], obuf)
        pltpu.sync_copy(obuf, out_hbm.at[cid].at[sl])

    @pl.when(sid == NUM_TILES - 1)
    def _():
        sl = pl.ds(TAIL_BASE, TAIL_ROWS)
        pltpu.sync_copy(acc.at[sl], obuf.at[pl.ds(0, TAIL_ROWS)])
        pltpu.sync_copy(obuf.at[pl.ds(0, TAIL_ROWS)], out_hbm.at[cid].at[sl])


def _add_body(a_ref, b_ref, o_ref):
    o_ref[...] = a_ref[...] + b_ref[...]


def kernel(adj_indices, adj_values, feature):
    row = adj_indices[0]
    col = adj_indices[1]
    pad = EPW_PAD - EPW
    colp = jnp.pad(col.reshape(NUM_WORKERS, EPW), ((0, 0), (0, pad)))
    rowp = jnp.pad(row.reshape(NUM_WORKERS, EPW), ((0, 0), (0, pad)))
    valp = jnp.pad(adj_values.reshape(NUM_WORKERS, EPW), ((0, 0), (0, pad)))
    pidx = jnp.stack(
        [colp.reshape(NUM_WORKERS, CHUNKS, K),
         rowp.reshape(NUM_WORKERS, CHUNKS, K)],
        axis=2).reshape(NUM_WORKERS * CHUNKS, 2, K)
    pval = valp.reshape(NUM_WORKERS * CHUNKS, K)

    mesh = plsc.VectorSubcoreMesh(
        core_axis_name="c", subcore_axis_name="s", num_cores=NUM_CORES)
    k = pl.kernel(
        _body,
        out_type=jax.ShapeDtypeStruct((NUM_CORES, N, D), jnp.float32),
        mesh=mesh,
        scratch_types=[
            pltpu.VMEM_SHARED((N, D), jnp.float32),
            pltpu.VMEM((2, K), jnp.int32),
            pltpu.VMEM((2, K), jnp.int32),
            pltpu.VMEM((K,), jnp.float32),
            pltpu.VMEM((K,), jnp.float32),
            pltpu.VMEM((K,), jnp.int32),
            pltpu.VMEM((K,), jnp.int32),
            pltpu.VMEM((K,), jnp.int32),
            pltpu.VMEM((K,), jnp.int32),
            pltpu.VMEM((K, D), jnp.float32),
            pltpu.VMEM((K, D), jnp.float32),
            pltpu.VMEM((OUT_CHUNK, D), jnp.float32),
            pltpu.SemaphoreType.DMA,
            pltpu.SemaphoreType.DMA,
            pltpu.SemaphoreType.DMA,
            pltpu.SemaphoreType.DMA,
            pltpu.SemaphoreType.DMA,
            pltpu.SemaphoreType.DMA,
        ],
    )
    out2 = k(pidx, pval, feature)

    blk = 2000
    return pl.pallas_call(
        _add_body,
        out_shape=jax.ShapeDtypeStruct((N, D), jnp.float32),
        grid=(N // blk,),
        in_specs=[pl.BlockSpec((blk, D), lambda i: (i, 0)),
                  pl.BlockSpec((blk, D), lambda i: (i, 0))],
        out_specs=pl.BlockSpec((blk, D), lambda i: (i, 0)),
    )(out2[0], out2[1])


# speedup vs baseline: 1.4518x; 1.3144x over previous
import jax
import jax.numpy as jnp
from jax import lax
from jax.experimental import pallas as pl
from jax.experimental.pallas import tpu as pltpu
from jax.experimental.pallas import tpu_sc as plsc

N = 10000
E = 320000
D = 128
LANES = 16

NUM_CORES = 2
NUM_TILES = 16
NUM_WORKERS = NUM_CORES * NUM_TILES
EPW = E // NUM_WORKERS
K = 80
EPW_PAD = 10080
CHUNKS = EPW_PAD // K
ROWS_PER_TILE = 624
OUT_CHUNK = 104
OUT_CHUNKS = ROWS_PER_TILE // OUT_CHUNK
TAIL_BASE = NUM_TILES * ROWS_PER_TILE
TAIL_ROWS = N - TAIL_BASE


def _body(pidx_hbm, pval_hbm, feat_hbm, out_hbm,
          acc, ebuf0, ebuf1, vbuf0, vbuf1, idxv0, idxv1, ridx0, ridx1,
          rows0, rows1, obuf, sem_e0, sem_e1, sem_g0, sem_g1, sem_s0, sem_s1):
    cid = lax.axis_index("c")
    sid = lax.axis_index("s")
    wid = cid * NUM_TILES + sid
    chunk_base = wid * CHUNKS

    def zrow(r, c):
        for j in range(D // LANES):
            obuf[r, pl.ds(LANES * j, LANES)] = jnp.zeros((LANES,), jnp.float32)
        return c
    lax.fori_loop(0, OUT_CHUNK, zrow, 0)
    row_base = sid * ROWS_PER_TILE
    for c in range(OUT_CHUNKS):
        pltpu.sync_copy(obuf, acc.at[pl.ds(row_base + c * OUT_CHUNK, OUT_CHUNK)])

    @pl.when(sid == NUM_TILES - 1)
    def _():
        pltpu.sync_copy(obuf.at[pl.ds(0, TAIL_ROWS)],
                        acc.at[pl.ds(TAIL_BASE, TAIL_ROWS)])
    plsc.subcore_barrier()

    def e_start(c, ebuf, vbuf, sem_e):
        pltpu.async_copy(pidx_hbm.at[chunk_base + c], ebuf, sem_e)
        pltpu.async_copy(pval_hbm.at[chunk_base + c], vbuf, sem_e)

    def e_wait(c, ebuf, vbuf, sem_e):
        pltpu.make_async_copy(pidx_hbm.at[chunk_base + c], ebuf, sem_e).wait()
        pltpu.make_async_copy(pval_hbm.at[chunk_base + c], vbuf, sem_e).wait()

    def g_start(idx_v, rows_v, sem_g):
        pltpu.async_copy(feat_hbm.at[idx_v], rows_v, sem_g)

    def g_wait(idx_v, rows_v, sem_g):
        pltpu.make_async_copy(feat_hbm.at[idx_v], rows_v, sem_g).wait()

    def s_start(ridx_v, rows_v, sem_s):
        pltpu.async_copy(rows_v, acc.at[ridx_v], sem_s, add=True)

    def s_wait(ridx_v, rows_v, sem_s):
        pltpu.make_async_copy(rows_v, acc.at[ridx_v], sem_s).wait()

    def scale(vbuf, rows_v):
        def e_body(t, cc):
            vv = vbuf[pl.ds(t * LANES, LANES)]
            for i in range(LANES):
                e = t * LANES + i
                v = vv[i]
                for j in range(D // LANES):
                    rows_v[e, pl.ds(LANES * j, LANES)] = (
                        rows_v[e, pl.ds(LANES * j, LANES)] * v)
            return cc
        lax.fori_loop(0, K // LANES, e_body, 0)

    bufs0 = (ebuf0, vbuf0, idxv0, ridx0, rows0, sem_e0, sem_g0, sem_s0)
    bufs1 = (ebuf1, vbuf1, idxv1, ridx1, rows1, sem_e1, sem_g1, sem_s1)

    def phase(c, A, B, drain_prev, prefetch_next):
        ebuf_a, vbuf_a, idxv_a, ridx_a, rows_a, sem_ea, sem_ga, sem_sa = A
        ebuf_b, vbuf_b, idxv_b, ridx_b, rows_b, sem_eb, sem_gb, sem_sb = B
        g_wait(idxv_a, rows_a, sem_ga)
        if drain_prev:
            s_wait(ridx_b, rows_b, sem_sb)
        if prefetch_next:
            e_start(c + 1, ebuf_b, vbuf_b, sem_eb)
        for t in range(K // LANES):
            ridx_a[pl.ds(t * LANES, LANES)] = ebuf_a[1, pl.ds(t * LANES, LANES)]
        scale(vbuf_a, rows_a)
        if prefetch_next:
            e_wait(c + 1, ebuf_b, vbuf_b, sem_eb)
            for t in range(K // LANES):
                idxv_b[pl.ds(t * LANES, LANES)] = ebuf_b[0, pl.ds(t * LANES, LANES)]
            g_start(idxv_b, rows_b, sem_gb)
        s_start(ridx_a, rows_a, sem_sa)

    e_start(0, ebuf0, vbuf0, sem_e0)
    e_wait(0, ebuf0, vbuf0, sem_e0)
    for t in range(K // LANES):
        idxv0[pl.ds(t * LANES, LANES)] = ebuf0[0, pl.ds(t * LANES, LANES)]
    g_start(idxv0, rows0, sem_g0)
    phase(0, bufs0, bufs1, False, True)
    phase(1, bufs1, bufs0, True, True)

    def pair_body(p, carry):
        c0 = 2 * p + 2
        phase(c0, bufs0, bufs1, True, True)
        phase(c0 + 1, bufs1, bufs0, True, True)
        return carry

    lax.fori_loop(0, (CHUNKS - 4) // 2, pair_body, 0)
    phase(CHUNKS - 2, bufs0, bufs1, True, True)
    phase(CHUNKS - 1, bufs1, bufs0, True, False)
    s_wait(ridx1, rows1, sem_s1)
    plsc.subcore_barrier()

    for c in range(OUT_CHUNKS):
        sl = pl.ds(row_base + c * OUT_CHUNK, OUT_CHUNK)
        pltpu.sync_copy(acc.at[sl], obuf)
        pltpu.sync_copy(obuf, out_hbm.at[cid].at[sl])

    @pl.when(sid == NUM_TILES - 1)
    def _():
        sl = pl.ds(TAIL_BASE, TAIL_ROWS)
        pltpu.sync_copy(acc.at[sl], obuf.at[pl.ds(0, TAIL_ROWS)])
        pltpu.sync_copy(obuf.at[pl.ds(0, TAIL_ROWS)], out_hbm.at[cid].at[sl])


def _add_body(a_ref, b_ref, o_ref):
    o_ref[...] = a_ref[...] + b_ref[...]


def kernel(adj_indices, adj_values, feature):
    row = adj_indices[0]
    col = adj_indices[1]
    pad = EPW_PAD - EPW
    colp = jnp.pad(col.reshape(NUM_WORKERS, EPW), ((0, 0), (0, pad)))
    rowp = jnp.pad(row.reshape(NUM_WORKERS, EPW), ((0, 0), (0, pad)))
    valp = jnp.pad(adj_values.reshape(NUM_WORKERS, EPW), ((0, 0), (0, pad)))
    pidx = jnp.stack(
        [colp.reshape(NUM_WORKERS, CHUNKS, K),
         rowp.reshape(NUM_WORKERS, CHUNKS, K)],
        axis=2).reshape(NUM_WORKERS * CHUNKS, 2, K)
    pval = valp.reshape(NUM_WORKERS * CHUNKS, K)

    mesh = plsc.VectorSubcoreMesh(
        core_axis_name="c", subcore_axis_name="s", num_cores=NUM_CORES)
    k = pl.kernel(
        _body,
        out_type=jax.ShapeDtypeStruct((NUM_CORES, N, D), jnp.float32),
        mesh=mesh,
        scratch_types=[
            pltpu.VMEM_SHARED((N, D), jnp.float32),
            pltpu.VMEM((2, K), jnp.int32),
            pltpu.VMEM((2, K), jnp.int32),
            pltpu.VMEM((K,), jnp.float32),
            pltpu.VMEM((K,), jnp.float32),
            pltpu.VMEM((K,), jnp.int32),
            pltpu.VMEM((K,), jnp.int32),
            pltpu.VMEM((K,), jnp.int32),
            pltpu.VMEM((K,), jnp.int32),
            pltpu.VMEM((K, D), jnp.float32),
            pltpu.VMEM((K, D), jnp.float32),
            pltpu.VMEM((OUT_CHUNK, D), jnp.float32),
            pltpu.SemaphoreType.DMA,
            pltpu.SemaphoreType.DMA,
            pltpu.SemaphoreType.DMA,
            pltpu.SemaphoreType.DMA,
            pltpu.SemaphoreType.DMA,
            pltpu.SemaphoreType.DMA,
        ],
    )
    out2 = k(pidx, pval, feature)

    blk = 2000
    return pl.pallas_call(
        _add_body,
        out_shape=jax.ShapeDtypeStruct((N, D), jnp.float32),
        grid=(N // blk,),
        in_specs=[pl.BlockSpec((blk, D), lambda i: (i, 0)),
                  pl.BlockSpec((blk, D), lambda i: (i, 0))],
        out_specs=pl.BlockSpec((blk, D), lambda i: (i, 0)),
    )(out2[0], out2[1])


# speedup vs baseline: 2.6893x; 1.8524x over previous
import jax
import jax.numpy as jnp
from jax import lax
from jax.experimental import pallas as pl
from jax.experimental.pallas import tpu as pltpu
from jax.experimental.pallas import tpu_sc as plsc

N = 10000
E = 320000
D = 128
LANES = 16

NUM_CORES = 2
NUM_TILES = 16
NUM_WORKERS = NUM_CORES * NUM_TILES
EPW = E // NUM_WORKERS
K = 80
CHUNKS = EPW // K
ROWS_PER_TILE = 624
OUT_CHUNK = 104
OUT_CHUNKS = ROWS_PER_TILE // OUT_CHUNK
TAIL_BASE = NUM_TILES * ROWS_PER_TILE
TAIL_ROWS = N - TAIL_BASE


def _body(row_hbm, col_hbm, val_hbm, feat_hbm, out_hbm,
          acc, idx0, idx1, idx2, ridx0, ridx1, ridx2, val0, val1, val2,
          rows0, rows1, rows2, obuf,
          sem_e0, sem_e1, sem_e2, sem_g0, sem_g1, sem_g2,
          sem_s0, sem_s1, sem_s2):
    cid = lax.axis_index("c")
    sid = lax.axis_index("s")
    wid = cid * NUM_TILES + sid
    edge_base = wid * EPW

    def zrow(r, c):
        for j in range(D // LANES):
            obuf[r, pl.ds(LANES * j, LANES)] = jnp.zeros((LANES,), jnp.float32)
        return c
    lax.fori_loop(0, OUT_CHUNK, zrow, 0)
    row_base = sid * ROWS_PER_TILE
    for c in range(OUT_CHUNKS):
        pltpu.sync_copy(obuf, acc.at[pl.ds(row_base + c * OUT_CHUNK, OUT_CHUNK)])

    @pl.when(sid == NUM_TILES - 1)
    def _():
        pltpu.sync_copy(obuf.at[pl.ds(0, TAIL_ROWS)],
                        acc.at[pl.ds(TAIL_BASE, TAIL_ROWS)])
    plsc.subcore_barrier()

    def e_start(c, S):
        idx_v, ridx_v, val_v, rows_v, sem_e, sem_g, sem_s = S
        base = edge_base + c * K
        pltpu.async_copy(col_hbm.at[pl.ds(base, K)], idx_v, sem_e)
        pltpu.async_copy(row_hbm.at[pl.ds(base, K)], ridx_v, sem_e)
        pltpu.async_copy(val_hbm.at[pl.ds(base, K)], val_v, sem_e)

    def e_wait(c, S):
        idx_v, ridx_v, val_v, rows_v, sem_e, sem_g, sem_s = S
        base = edge_base + c * K
        pltpu.make_async_copy(col_hbm.at[pl.ds(base, K)], idx_v, sem_e).wait()
        pltpu.make_async_copy(row_hbm.at[pl.ds(base, K)], ridx_v, sem_e).wait()
        pltpu.make_async_copy(val_hbm.at[pl.ds(base, K)], val_v, sem_e).wait()

    def g_start(S):
        idx_v, ridx_v, val_v, rows_v, sem_e, sem_g, sem_s = S
        pltpu.async_copy(feat_hbm.at[idx_v], rows_v, sem_g)

    def g_wait(S):
        idx_v, ridx_v, val_v, rows_v, sem_e, sem_g, sem_s = S
        pltpu.make_async_copy(feat_hbm.at[idx_v], rows_v, sem_g).wait()

    def s_start(S):
        idx_v, ridx_v, val_v, rows_v, sem_e, sem_g, sem_s = S
        pltpu.async_copy(rows_v, acc.at[ridx_v], sem_s, add=True)

    def s_wait(S):
        idx_v, ridx_v, val_v, rows_v, sem_e, sem_g, sem_s = S
        pltpu.make_async_copy(rows_v, acc.at[ridx_v], sem_s).wait()

    def scale(S):
        idx_v, ridx_v, val_v, rows_v, sem_e, sem_g, sem_s = S

        def e_body(t, cc):
            vv = val_v[pl.ds(t * LANES, LANES)]
            for i in range(LANES):
                e = t * LANES + i
                v = vv[i]
                for j in range(D // LANES):
                    rows_v[e, pl.ds(LANES * j, LANES)] = (
                        rows_v[e, pl.ds(LANES * j, LANES)] * v)
            return cc
        lax.fori_loop(0, K // LANES, e_body, 0)

    sets = [
        (idx0, ridx0, val0, rows0, sem_e0, sem_g0, sem_s0),
        (idx1, ridx1, val1, rows1, sem_e1, sem_g1, sem_s1),
        (idx2, ridx2, val2, rows2, sem_e2, sem_g2, sem_s2),
    ]

    def phase(c, X, Z, drain_prev, prefetch):
        g_wait(X)
        if drain_prev:
            s_wait(Z)
        if prefetch:
            e_start(c + 2, Z)
        scale(X)
        if prefetch:
            e_wait(c + 2, Z)
            g_start(Z)
        s_start(X)

    e_start(0, sets[0])
    e_start(1, sets[1])
    e_wait(0, sets[0])
    g_start(sets[0])
    e_wait(1, sets[1])
    g_start(sets[1])
    phase(0, sets[0], sets[2], False, True)
    phase(1, sets[1], sets[0], True, True)

    def triple_body(p, carry):
        c0 = 3 * p + 2
        phase(c0, sets[2], sets[1], True, True)
        phase(c0 + 1, sets[0], sets[2], True, True)
        phase(c0 + 2, sets[1], sets[0], True, True)
        return carry

    lax.fori_loop(0, (CHUNKS - 5) // 3, triple_body, 0)
    phase(CHUNKS - 3, sets[2], sets[1], True, True)
    phase(CHUNKS - 2, sets[0], sets[2], True, False)
    phase(CHUNKS - 1, sets[1], sets[0], True, False)
    s_wait(sets[1])
    plsc.subcore_barrier()

    for c in range(OUT_CHUNKS):
        sl = pl.ds(row_base + c * OUT_CHUNK, OUT_CHUNK)
        pltpu.sync_copy(acc.at[sl], obuf)
        pltpu.sync_copy(obuf, out_hbm.at[cid].at[sl])

    @pl.when(sid == NUM_TILES - 1)
    def _():
        sl = pl.ds(TAIL_BASE, TAIL_ROWS)
        pltpu.sync_copy(acc.at[sl], obuf.at[pl.ds(0, TAIL_ROWS)])
        pltpu.sync_copy(obuf.at[pl.ds(0, TAIL_ROWS)], out_hbm.at[cid].at[sl])


def _add_body(a_ref, b_ref, o_ref):
    o_ref[...] = a_ref[...] + b_ref[...]


def kernel(adj_indices, adj_values, feature):
    row = adj_indices[0]
    col = adj_indices[1]
    mesh = plsc.VectorSubcoreMesh(
        core_axis_name="c", subcore_axis_name="s", num_cores=NUM_CORES)
    k = pl.kernel(
        _body,
        out_type=jax.ShapeDtypeStruct((NUM_CORES, N, D), jnp.float32),
        mesh=mesh,
        scratch_types=[
            pltpu.VMEM_SHARED((N, D), jnp.float32),
            pltpu.VMEM((K,), jnp.int32),
            pltpu.VMEM((K,), jnp.int32),
            pltpu.VMEM((K,), jnp.int32),
            pltpu.VMEM((K,), jnp.int32),
            pltpu.VMEM((K,), jnp.int32),
            pltpu.VMEM((K,), jnp.int32),
            pltpu.VMEM((K,), jnp.float32),
            pltpu.VMEM((K,), jnp.float32),
            pltpu.VMEM((K,), jnp.float32),
            pltpu.VMEM((K, D), jnp.float32),
            pltpu.VMEM((K, D), jnp.float32),
            pltpu.VMEM((K, D), jnp.float32),
            pltpu.VMEM((OUT_CHUNK, D), jnp.float32),
            pltpu.SemaphoreType.DMA,
            pltpu.SemaphoreType.DMA,
            pltpu.SemaphoreType.DMA,
            pltpu.SemaphoreType.DMA,
            pltpu.SemaphoreType.DMA,
            pltpu.SemaphoreType.DMA,
            pltpu.SemaphoreType.DMA,
            pltpu.SemaphoreType.DMA,
            pltpu.SemaphoreType.DMA,
        ],
    )
    out2 = k(row, col, adj_values, feature)

    blk = 2000
    return pl.pallas_call(
        _add_body,
        out_shape=jax.ShapeDtypeStruct((N, D), jnp.float32),
        grid=(N // blk,),
        in_specs=[pl.BlockSpec((blk, D), lambda i: (i, 0)),
                  pl.BlockSpec((blk, D), lambda i: (i, 0))],
        out_specs=pl.BlockSpec((blk, D), lambda i: (i, 0)),
    )(out2[0], out2[1])
